# baseline (speedup 1.0000x reference)
import jax
import jax.numpy as jnp
from jax import lax
from jax.experimental import pallas as pl

_ALPHA = 0.95


def _row_sum_exact_order(x):
    r, c = x.shape
    xt = x.T
    acc = xt[0:8, :]
    for k in range(1, c // 8):
        acc = acc + xt[8 * k : 8 * k + 8, :]
    a = [acc[i : i + 1, :] for i in range(8)]
    rs_t = ((a[0] + a[4]) + (a[2] + a[6])) + ((a[1] + a[5]) + (a[3] + a[7]))
    return rs_t.T


def _loca_body(x_ref, lab_ref, out_ref):
    x = x_ref[...]
    lab = lab_ref[...]
    r, c = x.shape
    col = lax.broadcasted_iota(jnp.int32, (r, c), 1)
    mask = col == lab
    rs = _row_sum_exact_order(x)
    t = jnp.sum(jnp.where(mask, x, 0.0), axis=1, keepdims=True)
    s = _ALPHA / (1.0 - 2.0 * t + rs)
    tv = 1.0 - s * rs + s * t
    out_ref[...] = jnp.where(mask, tv, s * x)


def kernel(teacher_logits, true_labels):
    b, c = teacher_logits.shape
    rows = 1024
    lab2 = true_labels.astype(jnp.int32).reshape(b, 1)
    return pl.pallas_call(
        _loca_body,
        grid=(b // rows,),
        in_specs=[
            pl.BlockSpec((rows, c), lambda i: (i, 0)),
            pl.BlockSpec((rows, 1), lambda i: (i, 0)),
        ],
        out_specs=pl.BlockSpec((rows, c), lambda i: (i, 0)),
        out_shape=jax.ShapeDtypeStruct((b, c), jnp.float32),
    )(teacher_logits, lab2)
